# Initial kernel scaffold; baseline (speedup 1.0000x reference)
import jax
import jax.numpy as jnp
from jax.experimental import pallas as pl


def kernel(x, Wqk, Wv, Wout, bout, rotations):
    raise NotImplementedError("write your pallas kernel here")


# speedup vs baseline: 4.7015x; 4.7015x over previous
import jax
import jax.numpy as jnp
from jax.experimental import pallas as pl
from jax.experimental.pallas import tpu as pltpu
from jax.experimental.pallas import tpu_sc as plsc

B = 2
S = 4096
D = 1024
H = 16
DH = 64
NH = 8
NB = 64
BH = B * H
NI = NH * S
TOT = BH * NI
NCHUNK = NI // NB
ROW = 128
OROW = 128
TBLK = 256
CHUNK_BLK = 8
RANK_C = 256
SELF_VAL = -5e4


def _proj_hash_body(x_ref, w_ref, rot_ref, qkvt_ref, u_ref):
    i = pl.program_id(0)
    tb = i % (S // TBLK)
    xb = x_ref[...]
    qkv = jax.lax.dot_general(xb, w_ref[...], (((1,), (1,)), ((), ())),
                              preferred_element_type=jnp.float32)
    tcol = jax.lax.broadcasted_iota(jnp.int32, (TBLK, 1), 0) + tb * TBLK
    lane64 = jax.lax.broadcasted_iota(jnp.int32, (TBLK, NB), 1)
    for h in range(H):
        qk_h = qkv[:, DH * h:DH * h + DH]
        bits = jax.lax.bitcast_convert_type(qk_h[:, 0:1], jnp.int32)
        packed = jax.lax.bitcast_convert_type((bits & ~4095) | tcol,
                                              jnp.float32)
        qkvt_ref[h, :, 0:1] = packed
        qkvt_ref[h, :, 1:DH] = qk_h[:, 1:DH]
        qkvt_ref[h, :, DH:2 * DH] = qkv[:, D + DH * h:D + DH * h + DH]
        rot = jax.lax.dot_general(qk_h, rot_ref[...], (((1,), (0,)), ((), ())),
                                  preferred_element_type=jnp.float32)
        for hh in range(NH):
            r = rot[:, NB * hh:NB * hh + NB]
            m = jnp.max(r, axis=1, keepdims=True)
            idx = jnp.min(jnp.where(r == m, lane64, NB), axis=1, keepdims=True)
            u_ref[h, :, hh:hh + 1] = idx


def _proj_hash(x2, wcat, rot2):
    return pl.pallas_call(
        _proj_hash_body,
        grid=(B * S // TBLK,),
        in_specs=[
            pl.BlockSpec((TBLK, D), lambda i: (i, 0)),
            pl.BlockSpec((2 * D, D), lambda i: (0, 0)),
            pl.BlockSpec((DH, NH * NB), lambda i: (0, 0)),
        ],
        out_specs=[
            pl.BlockSpec((H, TBLK, ROW), lambda i: (i // (S // TBLK),
                                                    i % (S // TBLK), 0)),
            pl.BlockSpec((H, TBLK, NH), lambda i: (i // (S // TBLK),
                                                   i % (S // TBLK), 0)),
        ],
        out_shape=[
            jax.ShapeDtypeStruct((BH, S, ROW), jnp.float32),
            jax.ShapeDtypeStruct((BH, S, NH), jnp.int32),
        ],
    )(x2, wcat, rot2)


def _ranks_body(u_ref, sp_ref):
    bh = pl.program_id(0)
    base = (bh * NI).astype(jnp.float32)
    ub = u_ref[0]
    lane64 = jax.lax.broadcasted_iota(jnp.int32, (S, NB), 1)
    tri = (jax.lax.broadcasted_iota(jnp.int32, (RANK_C, RANK_C), 0)
           > jax.lax.broadcasted_iota(jnp.int32, (RANK_C, RANK_C), 1))
    trib = tri.astype(jnp.bfloat16)
    nchunks = S // RANK_C
    for h in range(NH):
        ob = (ub[:, h:h + 1] == lane64)
        obf = ob.astype(jnp.bfloat16)
        of = ob.astype(jnp.float32)
        run = jnp.zeros((1, NB), jnp.float32)
        parts = []
        for k in range(nchunks):
            sl = slice(RANK_C * k, RANK_C * (k + 1))
            pk = jax.lax.dot_general(trib, obf[sl], (((1,), (0,)), ((), ())),
                                     preferred_element_type=jnp.float32)
            parts.append(pk + run)
            run = run + jnp.sum(of[sl], axis=0, keepdims=True)
        padj = jnp.concatenate(parts, axis=0)
        rank = jnp.sum(padj * of, axis=1, keepdims=True)
        c = run
        for sft in (1, 2, 4, 8, 16, 32):
            c = c + jnp.concatenate(
                [jnp.zeros((1, sft), jnp.float32), c[:, :NB - sft]], axis=1)
        goff = c - run + float(h * S)
        sel = jnp.sum(of * goff, axis=1, keepdims=True)
        sp = sel + rank + base
        sp_ref[0, :, h:h + 1] = sp.astype(jnp.int32)


def _ranks(u):
    return pl.pallas_call(
        _ranks_body,
        grid=(BH,),
        in_specs=[pl.BlockSpec((1, S, NH), lambda i: (i, 0, 0))],
        out_specs=pl.BlockSpec((1, S, NH), lambda i: (i, 0, 0)),
        out_shape=jax.ShapeDtypeStruct((BH, S, NH), jnp.int32),
    )(u)


_SC_W = 128

def _sc_scatter(qkvt_flat, sp_flat):
    mesh = plsc.VectorSubcoreMesh(core_axis_name="core",
                                  subcore_axis_name="subcore")

    @pl.kernel(out_type=jax.ShapeDtypeStruct((TOT, ROW), jnp.float32),
               mesh=mesh, scratch_types=[])
    def k(x_hbm, i_hbm, o_hbm):
        def body(x_vmem, i_vmem):
            pltpu.sync_copy(x_vmem, o_hbm.at[i_vmem.at[0]])

        wins_per_bh = NI // _SC_W
        wins_per_seg = S // _SC_W
        pltpu.emit_pipeline(
            body,
            grid=(TOT // _SC_W,),
            in_specs=[
                pl.BlockSpec((_SC_W, ROW),
                             index_map=lambda w: (
                                 (w // wins_per_bh) * wins_per_seg
                                 + w % wins_per_seg, 0)),
                pl.BlockSpec((1, _SC_W), index_map=lambda w: (0, w)),
            ],
            out_specs=[],
            core_axis_name=("core", "subcore"),
            dimension_semantics=(pltpu.PARALLEL,),
        )(x_hbm, i_hbm)

    return k(qkvt_flat, sp_flat)


def _sc_gather(so_flat, sp_flat):
    mesh = plsc.VectorSubcoreMesh(core_axis_name="core",
                                  subcore_axis_name="subcore")

    @pl.kernel(out_type=jax.ShapeDtypeStruct((TOT, OROW), jnp.float32),
               mesh=mesh, scratch_types=[])
    def k(x_hbm, i_hbm, o_hbm):
        def body(i_vmem, o_vmem):
            pltpu.sync_copy(x_hbm.at[i_vmem.at[0]], o_vmem)

        pltpu.emit_pipeline(
            body,
            grid=(TOT // _SC_W,),
            in_specs=[pl.BlockSpec((1, _SC_W), index_map=lambda w: (0, w))],
            out_specs=[pl.BlockSpec((_SC_W, OROW),
                                    index_map=lambda w: (w, 0))],
            core_axis_name=("core", "subcore"),
            dimension_semantics=(pltpu.PARALLEL,),
        )(i_hbm, o_hbm)

    return k(so_flat, sp_flat)


def _attn_body(cur_ref, prev_ref, tbnd_ref, so_ref):
    cb = pl.program_id(1)
    is_bnd = (cb % (NCHUNK // NH // CHUNK_BLK)) == 0
    k_all = jnp.concatenate([prev_ref[0], cur_ref[0]], axis=0)
    tk_prev = tbnd_ref[0, 0]
    row64 = jax.lax.broadcasted_iota(jnp.int32, (NB, 1), 0)
    col128 = jax.lax.broadcasted_iota(jnp.int32, (1, 2 * NB), 1)
    diag = col128 == (row64 + NB)
    for k in range(CHUNK_BLK):
        qrows = k_all[NB * (k + 1):NB * (k + 2)]
        keys = k_all[NB * k:NB * (k + 2)]
        q = qrows[:, 0:DH]
        kk = keys[:, 0:DH]
        nrm = jnp.maximum(
            jnp.sqrt(jnp.sum(kk * kk, axis=1, keepdims=True)), 1e-12)
        bk = kk / nrm
        bv = keys[:, DH:2 * DH]
        dots = jax.lax.dot_general(q, bk, (((1,), (1,)), ((), ())),
                                   preferred_element_type=jnp.float32)
        dots = dots * (DH ** -0.5)
        dots = jnp.where(diag, SELF_VAL, dots)
        if k == 0:
            same_tok = jnp.logical_and(qrows[:, 0:1] == tk_prev, is_bnd)
            left = jnp.where(same_tok, SELF_VAL, dots[:, 0:NB])
            dots = jnp.concatenate([left, dots[:, NB:2 * NB]], axis=1)
        m = jnp.max(dots, axis=1, keepdims=True)
        e = jnp.exp(dots - m)
        ssum = jnp.sum(e, axis=1, keepdims=True)
        lse = m + jnp.log(ssum)
        bo = jax.lax.dot_general(e / ssum, bv, (((1,), (0,)), ((), ())),
                                 preferred_element_type=jnp.float32)
        so_ref[0, NB * k:NB * (k + 1), 0:DH] = bo
        so_ref[0, NB * k:NB * (k + 1), DH:DH + 1] = lse
        so_ref[0, NB * k:NB * (k + 1), DH + 1:OROW] = jnp.zeros(
            (NB, OROW - DH - 1), jnp.float32)


def _attention(sorted3, t_bnd):
    rows_blk = CHUNK_BLK * NB
    return pl.pallas_call(
        _attn_body,
        grid=(BH, NCHUNK // CHUNK_BLK),
        in_specs=[
            pl.BlockSpec((1, rows_blk, ROW), lambda bh, cb: (bh, cb, 0)),
            pl.BlockSpec((1, NB, ROW),
                         lambda bh, cb: (bh, (cb * CHUNK_BLK + NCHUNK - 1)
                                         % NCHUNK, 0)),
            pl.BlockSpec((1, 1, 1, NB),
                         lambda bh, cb: (bh, (cb // (NCHUNK // NH // CHUNK_BLK)
                                              + NH - 1) % NH, 0, 0)),
        ],
        out_specs=pl.BlockSpec((1, rows_blk, OROW),
                               lambda bh, cb: (bh, cb, 0)),
        out_shape=jax.ShapeDtypeStruct((BH, NI, OROW), jnp.float32),
    )(sorted3, sorted3, t_bnd)


def _combine_body(g_ref, wout_ref, bout_ref, out_ref):
    combs = []
    for h in range(H):
        gh = g_ref[h]
        logits = gh[:, :, DH:DH + 1]
        m = jnp.max(logits, axis=0)
        e = jnp.exp(logits - m[None])
        ssum = jnp.sum(e, axis=0)
        probs = e / ssum[None]
        combs.append(jnp.sum(gh[:, :, 0:DH] * probs, axis=0))
    out_tok = jnp.concatenate(combs, axis=1)
    res = jax.lax.dot_general(out_tok, wout_ref[...], (((1,), (1,)), ((), ())),
                              preferred_element_type=jnp.float32)
    out_ref[0] = res + bout_ref[...]


def _combine(gathered, wout, bout2):
    return pl.pallas_call(
        _combine_body,
        grid=(B, S // TBLK),
        in_specs=[
            pl.BlockSpec((H, NH, TBLK, OROW), lambda b, tb: (b, 0, tb, 0)),
            pl.BlockSpec((D, D), lambda b, tb: (0, 0)),
            pl.BlockSpec((1, D), lambda b, tb: (0, 0)),
        ],
        out_specs=pl.BlockSpec((1, TBLK, D), lambda b, tb: (b, tb, 0)),
        out_shape=jax.ShapeDtypeStruct((B, S, D), jnp.float32),
    )(gathered, wout, bout2)


def kernel(x, Wqk, Wv, Wout, bout, rotations):
    x2 = x.reshape(B * S, D)
    wcat = jnp.concatenate([Wqk, Wv], axis=0)
    rot2 = jnp.concatenate([rotations, -rotations], axis=2).reshape(DH, NH * NB)

    qkvt, u = _proj_hash(x2, wcat, rot2)
    sp = _ranks(u)
    sp_flat = sp.transpose(0, 2, 1).reshape(1, TOT)

    sorted_flat = _sc_scatter(qkvt.reshape(BH * S, ROW), sp_flat)
    sorted3 = sorted_flat.reshape(BH, NI, ROW)
    t_bnd = (sorted3.reshape(BH, NH, NCHUNK // NH, NB, ROW)
             [:, :, NCHUNK // NH - 1, :, 0].reshape(BH, NH, 1, NB))

    so = _attention(sorted3, t_bnd)
    gathered = _sc_gather(so.reshape(TOT, OROW), sp_flat)
    g4 = gathered.reshape(BH, NH, S, OROW)

    return _combine(g4, Wout, bout.reshape(1, D))


# speedup vs baseline: 7.6245x; 1.6217x over previous
import jax
import jax.numpy as jnp
from jax.experimental import pallas as pl
from jax.experimental.pallas import tpu as pltpu
from jax.experimental.pallas import tpu_sc as plsc

B = 2
S = 4096
D = 1024
H = 16
DH = 64
NH = 8
NB = 64
BH = B * H
NI = NH * S
TOT = BH * NI
NCHUNK = NI // NB
ROW = 128
OROW = 128
TBLK = 256
CHUNK_BLK = 8
RANK_C = 256
SELF_VAL = -5e4


def _proj_hash_body(x_ref, w_ref, rot_ref, qkvt_ref, u_ref):
    i = pl.program_id(0)
    tb = i % (S // TBLK)
    xb = x_ref[...]
    qkv = jax.lax.dot_general(xb, w_ref[...], (((1,), (1,)), ((), ())),
                              preferred_element_type=jnp.float32)
    tcol = jax.lax.broadcasted_iota(jnp.int32, (TBLK, 1), 0) + tb * TBLK
    lane64 = jax.lax.broadcasted_iota(jnp.int32, (TBLK, NB), 1)
    for h in range(H):
        qk_h = qkv[:, DH * h:DH * h + DH]
        bits = jax.lax.bitcast_convert_type(qk_h[:, 0:1], jnp.int32)
        packed = jax.lax.bitcast_convert_type((bits & ~4095) | tcol,
                                              jnp.float32)
        qkvt_ref[h, :, 0:1] = packed
        qkvt_ref[h, :, 1:DH] = qk_h[:, 1:DH]
        qkvt_ref[h, :, DH:2 * DH] = qkv[:, D + DH * h:D + DH * h + DH]
        rot = jax.lax.dot_general(qk_h, rot_ref[...], (((1,), (0,)), ((), ())),
                                  preferred_element_type=jnp.float32)
        for hh in range(NH):
            r = rot[:, NB * hh:NB * hh + NB]
            m = jnp.max(r, axis=1, keepdims=True)
            idx = jnp.min(jnp.where(r == m, lane64, NB), axis=1, keepdims=True)
            u_ref[h, :, hh:hh + 1] = idx


def _proj_hash(x2, wcat, rot2):
    return pl.pallas_call(
        _proj_hash_body,
        grid=(B * S // TBLK,),
        in_specs=[
            pl.BlockSpec((TBLK, D), lambda i: (i, 0)),
            pl.BlockSpec((2 * D, D), lambda i: (0, 0)),
            pl.BlockSpec((DH, NH * NB), lambda i: (0, 0)),
        ],
        out_specs=[
            pl.BlockSpec((H, TBLK, ROW), lambda i: (i // (S // TBLK),
                                                    i % (S // TBLK), 0)),
            pl.BlockSpec((H, TBLK, NH), lambda i: (i // (S // TBLK),
                                                   i % (S // TBLK), 0)),
        ],
        out_shape=[
            jax.ShapeDtypeStruct((BH, S, ROW), jnp.float32),
            jax.ShapeDtypeStruct((BH, S, NH), jnp.int32),
        ],
    )(x2, wcat, rot2)


def _ranks_body(u_ref, sp_ref):
    bh = pl.program_id(0)
    base = (bh * NI).astype(jnp.float32)
    ub = u_ref[0]
    nbins = NH * NB
    lanes = jax.lax.broadcasted_iota(jnp.int32, (S, nbins), 1)
    ob = (ub[:, 0:1] + 0 * NB) == lanes
    for h in range(1, NH):
        ob = jnp.logical_or(ob, (ub[:, h:h + 1] + h * NB) == lanes)
    obf = ob.astype(jnp.bfloat16)
    of = ob.astype(jnp.float32)
    tri = (jax.lax.broadcasted_iota(jnp.int32, (RANK_C, RANK_C), 0)
           > jax.lax.broadcasted_iota(jnp.int32, (RANK_C, RANK_C), 1))
    trib = tri.astype(jnp.bfloat16)
    nchunks = S // RANK_C
    csums = [jnp.sum(of[RANK_C * k:RANK_C * (k + 1)], axis=0, keepdims=True)
             for k in range(nchunks)]
    hist = csums[0]
    for k in range(1, nchunks):
        hist = hist + csums[k]
    c = hist
    for sft in (1, 2, 4, 8, 16, 32, 64, 128, 256):
        c = c + jnp.concatenate(
            [jnp.zeros((1, sft), jnp.float32), c[:, :nbins - sft]], axis=1)
    goff = c - hist
    run = jnp.zeros((1, nbins), jnp.float32)
    for k in range(nchunks):
        sl = slice(RANK_C * k, RANK_C * (k + 1))
        pk = jax.lax.dot_general(trib, obf[sl], (((1,), (0,)), ((), ())),
                                 preferred_element_type=jnp.float32)
        padj = pk + (run + goff)
        for h in range(NH):
            g = slice(h * NB, (h + 1) * NB)
            val = jnp.sum(padj[:, g] * of[sl, g], axis=1, keepdims=True)
            sp_ref[0, sl, h:h + 1] = (val + base).astype(jnp.int32)
        run = run + csums[k]


def _ranks(u):
    return pl.pallas_call(
        _ranks_body,
        grid=(BH,),
        in_specs=[pl.BlockSpec((1, S, NH), lambda i: (i, 0, 0))],
        out_specs=pl.BlockSpec((1, S, NH), lambda i: (i, 0, 0)),
        out_shape=jax.ShapeDtypeStruct((BH, S, NH), jnp.int32),
    )(u)


_SC_W = 128

def _sc_scatter(qkvt_flat, sp_flat):
    mesh = plsc.VectorSubcoreMesh(core_axis_name="core",
                                  subcore_axis_name="subcore")

    @pl.kernel(out_type=jax.ShapeDtypeStruct((TOT, ROW), jnp.float32),
               mesh=mesh, scratch_types=[])
    def k(x_hbm, i_hbm, o_hbm):
        def body(x_vmem, i_vmem):
            pltpu.sync_copy(x_vmem, o_hbm.at[i_vmem.at[0]])

        wins_per_bh = NI // _SC_W
        wins_per_seg = S // _SC_W
        pltpu.emit_pipeline(
            body,
            grid=(TOT // _SC_W,),
            in_specs=[
                pl.BlockSpec((_SC_W, ROW),
                             index_map=lambda w: (
                                 (w // wins_per_bh) * wins_per_seg
                                 + w % wins_per_seg, 0)),
                pl.BlockSpec((1, _SC_W), index_map=lambda w: (0, w)),
            ],
            out_specs=[],
            core_axis_name=("core", "subcore"),
            dimension_semantics=(pltpu.PARALLEL,),
        )(x_hbm, i_hbm)

    return k(qkvt_flat, sp_flat)


def _sc_gather(so_flat, sp_flat):
    mesh = plsc.VectorSubcoreMesh(core_axis_name="core",
                                  subcore_axis_name="subcore")

    @pl.kernel(out_type=jax.ShapeDtypeStruct((TOT, OROW), jnp.float32),
               mesh=mesh, scratch_types=[])
    def k(x_hbm, i_hbm, o_hbm):
        def body(i_vmem, o_vmem):
            pltpu.sync_copy(x_hbm.at[i_vmem.at[0]], o_vmem)

        pltpu.emit_pipeline(
            body,
            grid=(TOT // _SC_W,),
            in_specs=[pl.BlockSpec((1, _SC_W), index_map=lambda w: (0, w))],
            out_specs=[pl.BlockSpec((_SC_W, OROW),
                                    index_map=lambda w: (w, 0))],
            core_axis_name=("core", "subcore"),
            dimension_semantics=(pltpu.PARALLEL,),
        )(i_hbm, o_hbm)

    return k(so_flat, sp_flat)


def _attn_body(cur_ref, prev_ref, tbnd_ref, so_ref):
    cb = pl.program_id(1)
    is_bnd = (cb % (NCHUNK // NH // CHUNK_BLK)) == 0
    nq = CHUNK_BLK * NB
    nk = nq + NB
    k_all = jnp.concatenate([prev_ref[0], cur_ref[0]], axis=0)
    kk = k_all[:, 0:DH]
    nrm = jnp.maximum(
        jnp.sqrt(jnp.sum(kk * kk, axis=1, keepdims=True)), 1e-12)
    bk = kk / nrm
    q = k_all[NB:, 0:DH]
    dots = jax.lax.dot_general(q, bk, (((1,), (1,)), ((), ())),
                               preferred_element_type=jnp.float32)
    dots = dots * (DH ** -0.5)
    rc = jax.lax.broadcasted_iota(jnp.int32, (nq, 1), 0) // NB
    cc = jax.lax.broadcasted_iota(jnp.int32, (1, nk), 1) // NB
    d = cc - rc
    in_band = jnp.logical_and(d >= 0, d <= 1)
    ri = jax.lax.broadcasted_iota(jnp.int32, (nq, 1), 0)
    ci = jax.lax.broadcasted_iota(jnp.int32, (1, nk), 1)
    diag = ci == (ri + NB)
    dots = jnp.where(diag, SELF_VAL, dots)
    tk_prev = tbnd_ref[0, 0]
    same_tok = jnp.logical_and(k_all[NB:, 0:1] == tk_prev, is_bnd)
    same_tok = jnp.logical_and(same_tok, ri < NB)
    left = jnp.where(same_tok, SELF_VAL, dots[:, 0:NB])
    dots = jnp.concatenate([left, dots[:, NB:]], axis=1)
    dots = jnp.where(in_band, dots, -jnp.inf)
    m = jnp.max(dots, axis=1, keepdims=True)
    e = jnp.exp(dots - m)
    ssum = jnp.sum(e, axis=1, keepdims=True)
    lse = m + jnp.log(ssum)
    bo = jax.lax.dot_general(e / ssum, k_all[:, DH:2 * DH],
                             (((1,), (0,)), ((), ())),
                             preferred_element_type=jnp.float32)
    so_ref[0, :, 0:DH] = bo
    so_ref[0, :, DH:DH + 1] = lse
    so_ref[0, :, DH + 1:OROW] = jnp.zeros((nq, OROW - DH - 1), jnp.float32)


def _attention(sorted3, t_bnd):
    rows_blk = CHUNK_BLK * NB
    return pl.pallas_call(
        _attn_body,
        grid=(BH, NCHUNK // CHUNK_BLK),
        in_specs=[
            pl.BlockSpec((1, rows_blk, ROW), lambda bh, cb: (bh, cb, 0)),
            pl.BlockSpec((1, NB, ROW),
                         lambda bh, cb: (bh, (cb * CHUNK_BLK + NCHUNK - 1)
                                         % NCHUNK, 0)),
            pl.BlockSpec((1, 1, 1, NB),
                         lambda bh, cb: (bh, (cb // (NCHUNK // NH // CHUNK_BLK)
                                              + NH - 1) % NH, 0, 0)),
        ],
        out_specs=pl.BlockSpec((1, rows_blk, OROW),
                               lambda bh, cb: (bh, cb, 0)),
        out_shape=jax.ShapeDtypeStruct((BH, NI, OROW), jnp.float32),
    )(sorted3, sorted3, t_bnd)


def _combine_body(g_ref, wout_ref, bout_ref, out_ref):
    combs = []
    for h in range(H):
        gh = g_ref[h]
        logits = gh[:, :, DH:DH + 1]
        m = jnp.max(logits, axis=0)
        e = jnp.exp(logits - m[None])
        ssum = jnp.sum(e, axis=0)
        probs = e / ssum[None]
        combs.append(jnp.sum(gh[:, :, 0:DH] * probs, axis=0))
    out_tok = jnp.concatenate(combs, axis=1)
    res = jax.lax.dot_general(out_tok, wout_ref[...], (((1,), (1,)), ((), ())),
                              preferred_element_type=jnp.float32)
    out_ref[0] = res + bout_ref[...]


def _combine(gathered, wout, bout2):
    return pl.pallas_call(
        _combine_body,
        grid=(B, S // TBLK),
        in_specs=[
            pl.BlockSpec((H, NH, TBLK, OROW), lambda b, tb: (b, 0, tb, 0)),
            pl.BlockSpec((D, D), lambda b, tb: (0, 0)),
            pl.BlockSpec((1, D), lambda b, tb: (0, 0)),
        ],
        out_specs=pl.BlockSpec((1, TBLK, D), lambda b, tb: (b, tb, 0)),
        out_shape=jax.ShapeDtypeStruct((B, S, D), jnp.float32),
    )(gathered, wout, bout2)


def kernel(x, Wqk, Wv, Wout, bout, rotations):
    x2 = x.reshape(B * S, D)
    wcat = jnp.concatenate([Wqk, Wv], axis=0)
    rot2 = jnp.concatenate([rotations, -rotations], axis=2).reshape(DH, NH * NB)

    qkvt, u = _proj_hash(x2, wcat, rot2)
    sp = _ranks(u)
    sp_flat = sp.transpose(0, 2, 1).reshape(1, TOT)

    sorted_flat = _sc_scatter(qkvt.reshape(BH * S, ROW), sp_flat)
    sorted3 = sorted_flat.reshape(BH, NI, ROW)
    t_bnd = (sorted3.reshape(BH, NH, NCHUNK // NH, NB, ROW)
             [:, :, NCHUNK // NH - 1, :, 0].reshape(BH, NH, 1, NB))

    so = _attention(sorted3, t_bnd)
    gathered = _sc_gather(so.reshape(TOT, OROW), sp_flat)
    g4 = gathered.reshape(BH, NH, S, OROW)

    return _combine(g4, Wout, bout.reshape(1, D))


# speedup vs baseline: 7.8703x; 1.0322x over previous
import jax
import jax.numpy as jnp
from jax.experimental import pallas as pl
from jax.experimental.pallas import tpu as pltpu
from jax.experimental.pallas import tpu_sc as plsc

B = 2
S = 4096
D = 1024
H = 16
DH = 64
NH = 8
NB = 64
BH = B * H
NI = NH * S
TOT = BH * NI
NCHUNK = NI // NB
ROW = 128
OROW = 128
TBLK = 256
CHUNK_BLK = 8
RANK_C = 256
SELF_VAL = -5e4


def _proj_hash_body(x_ref, w_ref, rot_ref, qkvt_ref, u_ref):
    i = pl.program_id(0)
    tb = i % (S // TBLK)
    xb = x_ref[...]
    qkv = jax.lax.dot_general(xb, w_ref[...], (((1,), (1,)), ((), ())),
                              preferred_element_type=jnp.float32)
    tcol = jax.lax.broadcasted_iota(jnp.int32, (TBLK, 1), 0) + tb * TBLK
    lane64 = jax.lax.broadcasted_iota(jnp.int32, (TBLK, NB), 1)
    for h in range(H):
        qk_h = qkv[:, DH * h:DH * h + DH]
        bits = jax.lax.bitcast_convert_type(qk_h[:, 0:1], jnp.int32)
        packed = jax.lax.bitcast_convert_type((bits & ~4095) | tcol,
                                              jnp.float32)
        qkvt_ref[h, :, 0:1] = packed
        qkvt_ref[h, :, 1:DH] = qk_h[:, 1:DH]
        qkvt_ref[h, :, DH:2 * DH] = qkv[:, D + DH * h:D + DH * h + DH]
        rot = jax.lax.dot_general(qk_h, rot_ref[...], (((1,), (0,)), ((), ())),
                                  preferred_element_type=jnp.float32)
        for hh in range(NH):
            r = rot[:, NB * hh:NB * hh + NB]
            m = jnp.max(r, axis=1, keepdims=True)
            idx = jnp.min(jnp.where(r == m, lane64, NB), axis=1, keepdims=True)
            u_ref[h, :, hh:hh + 1] = idx


def _proj_hash(x2, wcat, rot2):
    return pl.pallas_call(
        _proj_hash_body,
        grid=(B * S // TBLK,),
        in_specs=[
            pl.BlockSpec((TBLK, D), lambda i: (i, 0)),
            pl.BlockSpec((2 * D, D), lambda i: (0, 0)),
            pl.BlockSpec((DH, NH * NB), lambda i: (0, 0)),
        ],
        out_specs=[
            pl.BlockSpec((H, TBLK, ROW), lambda i: (i // (S // TBLK),
                                                    i % (S // TBLK), 0)),
            pl.BlockSpec((H, TBLK, NH), lambda i: (i // (S // TBLK),
                                                   i % (S // TBLK), 0)),
        ],
        out_shape=[
            jax.ShapeDtypeStruct((BH, S, ROW), jnp.float32),
            jax.ShapeDtypeStruct((BH, S, NH), jnp.int32),
        ],
    )(x2, wcat, rot2)


def _ranks_body(u_ref, sp_ref):
    bh = pl.program_id(0)
    base = (bh * NI).astype(jnp.float32)
    ub = u_ref[0]
    nbins = NH * NB
    lanes = jax.lax.broadcasted_iota(jnp.int32, (S, nbins), 1)
    ob = (ub[:, 0:1] + 0 * NB) == lanes
    for h in range(1, NH):
        ob = jnp.logical_or(ob, (ub[:, h:h + 1] + h * NB) == lanes)
    obf = ob.astype(jnp.bfloat16)
    of = ob.astype(jnp.float32)
    tri = (jax.lax.broadcasted_iota(jnp.int32, (RANK_C, RANK_C), 0)
           > jax.lax.broadcasted_iota(jnp.int32, (RANK_C, RANK_C), 1))
    trib = tri.astype(jnp.bfloat16)
    ones_row = jnp.ones((1, RANK_C), jnp.bfloat16)
    gsel = (jax.lax.broadcasted_iota(jnp.int32, (nbins, NH), 0) // NB
            == jax.lax.broadcasted_iota(jnp.int32, (nbins, NH), 1)
            ).astype(jnp.bfloat16)
    nchunks = S // RANK_C
    csums = [jax.lax.dot_general(ones_row, obf[RANK_C * k:RANK_C * (k + 1)],
                                 (((1,), (0,)), ((), ())),
                                 preferred_element_type=jnp.float32)
             for k in range(nchunks)]
    hist = csums[0]
    for k in range(1, nchunks):
        hist = hist + csums[k]
    c = hist
    for sft in (1, 2, 4, 8, 16, 32, 64, 128, 256):
        c = c + jnp.concatenate(
            [jnp.zeros((1, sft), jnp.float32), c[:, :nbins - sft]], axis=1)
    goff = c - hist
    run = jnp.zeros((1, nbins), jnp.float32)
    for k in range(nchunks):
        sl = slice(RANK_C * k, RANK_C * (k + 1))
        pk = jax.lax.dot_general(trib, obf[sl], (((1,), (0,)), ((), ())),
                                 preferred_element_type=jnp.float32)
        padj = (pk + (run + goff)) * of[sl]
        hi = jnp.floor(padj * (1.0 / 256.0))
        lo = padj - hi * 256.0
        shi = jax.lax.dot_general(hi.astype(jnp.bfloat16), gsel,
                                  (((1,), (0,)), ((), ())),
                                  preferred_element_type=jnp.float32)
        slo = jax.lax.dot_general(lo.astype(jnp.bfloat16), gsel,
                                  (((1,), (0,)), ((), ())),
                                  preferred_element_type=jnp.float32)
        sp_ref[0, sl, :] = (shi * 256.0 + slo + base).astype(jnp.int32)
        run = run + csums[k]


def _ranks(u):
    return pl.pallas_call(
        _ranks_body,
        grid=(BH,),
        in_specs=[pl.BlockSpec((1, S, NH), lambda i: (i, 0, 0))],
        out_specs=pl.BlockSpec((1, S, NH), lambda i: (i, 0, 0)),
        out_shape=jax.ShapeDtypeStruct((BH, S, NH), jnp.int32),
    )(u)


_SC_W = 128

def _sc_scatter(qkvt_flat, sp_flat):
    mesh = plsc.VectorSubcoreMesh(core_axis_name="core",
                                  subcore_axis_name="subcore")

    @pl.kernel(out_type=jax.ShapeDtypeStruct((TOT, ROW), jnp.float32),
               mesh=mesh, scratch_types=[])
    def k(x_hbm, i_hbm, o_hbm):
        def body(x_vmem, i_vmem):
            pltpu.sync_copy(x_vmem, o_hbm.at[i_vmem.at[0]])

        wins_per_bh = NI // _SC_W
        wins_per_seg = S // _SC_W
        pltpu.emit_pipeline(
            body,
            grid=(TOT // _SC_W,),
            in_specs=[
                pl.BlockSpec((_SC_W, ROW),
                             index_map=lambda w: (
                                 (w // wins_per_bh) * wins_per_seg
                                 + w % wins_per_seg, 0)),
                pl.BlockSpec((1, _SC_W), index_map=lambda w: (0, w)),
            ],
            out_specs=[],
            core_axis_name=("core", "subcore"),
            dimension_semantics=(pltpu.PARALLEL,),
        )(x_hbm, i_hbm)

    return k(qkvt_flat, sp_flat)


def _sc_gather(so_flat, sp_flat):
    mesh = plsc.VectorSubcoreMesh(core_axis_name="core",
                                  subcore_axis_name="subcore")

    @pl.kernel(out_type=jax.ShapeDtypeStruct((TOT, OROW), jnp.float32),
               mesh=mesh, scratch_types=[])
    def k(x_hbm, i_hbm, o_hbm):
        def body(i_vmem, o_vmem):
            pltpu.sync_copy(x_hbm.at[i_vmem.at[0]], o_vmem)

        pltpu.emit_pipeline(
            body,
            grid=(TOT // _SC_W,),
            in_specs=[pl.BlockSpec((1, _SC_W), index_map=lambda w: (0, w))],
            out_specs=[pl.BlockSpec((_SC_W, OROW),
                                    index_map=lambda w: (w, 0))],
            core_axis_name=("core", "subcore"),
            dimension_semantics=(pltpu.PARALLEL,),
        )(i_hbm, o_hbm)

    return k(so_flat, sp_flat)


def _attn_body(cur_ref, prev_ref, tbnd_ref, so_ref):
    cb = pl.program_id(1)
    is_bnd = (cb % (NCHUNK // NH // CHUNK_BLK)) == 0
    nq = CHUNK_BLK * NB
    nk = nq + NB
    k_all = jnp.concatenate([prev_ref[0], cur_ref[0]], axis=0)
    kk = k_all[:, 0:DH]
    nrm = jnp.maximum(
        jnp.sqrt(jnp.sum(kk * kk, axis=1, keepdims=True)), 1e-12)
    bk = (kk / nrm).astype(jnp.bfloat16)
    q = k_all[NB:, 0:DH].astype(jnp.bfloat16)
    dots = jax.lax.dot_general(q, bk, (((1,), (1,)), ((), ())),
                               preferred_element_type=jnp.float32)
    dots = dots * (DH ** -0.5)
    rc = jax.lax.broadcasted_iota(jnp.int32, (nq, 1), 0) // NB
    cc = jax.lax.broadcasted_iota(jnp.int32, (1, nk), 1) // NB
    d = cc - rc
    in_band = jnp.logical_and(d >= 0, d <= 1)
    ri = jax.lax.broadcasted_iota(jnp.int32, (nq, 1), 0)
    ci = jax.lax.broadcasted_iota(jnp.int32, (1, nk), 1)
    diag = ci == (ri + NB)
    dots = jnp.where(diag, SELF_VAL, dots)
    tk_prev = tbnd_ref[0, 0]
    same_tok = jnp.logical_and(k_all[NB:, 0:1] == tk_prev, is_bnd)
    same_tok = jnp.logical_and(same_tok, ri < NB)
    left = jnp.where(same_tok, SELF_VAL, dots[:, 0:NB])
    dots = jnp.concatenate([left, dots[:, NB:]], axis=1)
    dots = jnp.where(in_band, dots, -jnp.inf)
    m = jnp.max(dots, axis=1, keepdims=True)
    e = jnp.exp(dots - m)
    ssum = jnp.sum(e, axis=1, keepdims=True)
    lse = m + jnp.log(ssum)
    bo = jax.lax.dot_general((e / ssum).astype(jnp.bfloat16),
                             k_all[:, DH:2 * DH].astype(jnp.bfloat16),
                             (((1,), (0,)), ((), ())),
                             preferred_element_type=jnp.float32)
    so_ref[0, :, 0:DH] = bo
    so_ref[0, :, DH:DH + 1] = lse


def _attention(sorted3, t_bnd):
    rows_blk = CHUNK_BLK * NB
    return pl.pallas_call(
        _attn_body,
        grid=(BH, NCHUNK // CHUNK_BLK),
        in_specs=[
            pl.BlockSpec((1, rows_blk, ROW), lambda bh, cb: (bh, cb, 0)),
            pl.BlockSpec((1, NB, ROW),
                         lambda bh, cb: (bh, (cb * CHUNK_BLK + NCHUNK - 1)
                                         % NCHUNK, 0)),
            pl.BlockSpec((1, 1, 1, NB),
                         lambda bh, cb: (bh, (cb // (NCHUNK // NH // CHUNK_BLK)
                                              + NH - 1) % NH, 0, 0)),
        ],
        out_specs=pl.BlockSpec((1, rows_blk, OROW),
                               lambda bh, cb: (bh, cb, 0)),
        out_shape=jax.ShapeDtypeStruct((BH, NI, OROW), jnp.float32),
    )(sorted3, sorted3, t_bnd)


def _combine_body(g_ref, wout_ref, bout_ref, out_ref):
    combs = []
    for h in range(H):
        gh = g_ref[h]
        logits = gh[:, :, DH:DH + 1]
        m = jnp.max(logits, axis=0)
        e = jnp.exp(logits - m[None])
        ssum = jnp.sum(e, axis=0)
        probs = e / ssum[None]
        combs.append(jnp.sum(gh[:, :, 0:DH] * probs, axis=0))
    out_tok = jnp.concatenate(combs, axis=1)
    res = jax.lax.dot_general(out_tok, wout_ref[...], (((1,), (1,)), ((), ())),
                              preferred_element_type=jnp.float32)
    out_ref[0] = res + bout_ref[...]


def _combine(gathered, wout, bout2):
    return pl.pallas_call(
        _combine_body,
        grid=(B, S // TBLK),
        in_specs=[
            pl.BlockSpec((H, NH, TBLK, OROW), lambda b, tb: (b, 0, tb, 0)),
            pl.BlockSpec((D, D), lambda b, tb: (0, 0)),
            pl.BlockSpec((1, D), lambda b, tb: (0, 0)),
        ],
        out_specs=pl.BlockSpec((1, TBLK, D), lambda b, tb: (b, tb, 0)),
        out_shape=jax.ShapeDtypeStruct((B, S, D), jnp.float32),
    )(gathered, wout, bout2)


def kernel(x, Wqk, Wv, Wout, bout, rotations):
    x2 = x.reshape(B * S, D)
    wcat = jnp.concatenate([Wqk, Wv], axis=0)
    rot2 = jnp.concatenate([rotations, -rotations], axis=2).reshape(DH, NH * NB)

    qkvt, u = _proj_hash(x2, wcat, rot2)
    sp = _ranks(u)
    sp_flat = sp.transpose(0, 2, 1).reshape(1, TOT)

    sorted_flat = _sc_scatter(qkvt.reshape(BH * S, ROW), sp_flat)
    sorted3 = sorted_flat.reshape(BH, NI, ROW)
    t_bnd = (sorted3.reshape(BH, NH, NCHUNK // NH, NB, ROW)
             [:, :, NCHUNK // NH - 1, :, 0].reshape(BH, NH, 1, NB))

    so = _attention(sorted3, t_bnd)
    gathered = _sc_gather(so.reshape(TOT, OROW), sp_flat)
    g4 = gathered.reshape(BH, NH, S, OROW)

    return _combine(g4, Wout, bout.reshape(1, D))


# speedup vs baseline: 8.8783x; 1.1281x over previous
import jax
import jax.numpy as jnp
from jax.experimental import pallas as pl
from jax.experimental.pallas import tpu as pltpu
from jax.experimental.pallas import tpu_sc as plsc

B = 2
S = 4096
D = 1024
H = 16
DH = 64
NH = 8
NB = 64
BH = B * H
NI = NH * S
TOT = BH * NI
NCHUNK = NI // NB
ROW = 128
OROW = 128
TBLK = 256
CHUNK_BLK = 8
RANK_C = 256
SELF_VAL = -5e4


def _proj_hash_body(x_ref, w_ref, rot_ref, qkvt_ref, u_ref):
    i = pl.program_id(0)
    tb = i % (S // TBLK)
    xb = x_ref[...]
    qkv = jax.lax.dot_general(xb, w_ref[...], (((1,), (1,)), ((), ())),
                              preferred_element_type=jnp.float32)
    tcol = jax.lax.broadcasted_iota(jnp.int32, (TBLK, 1), 0) + tb * TBLK
    wlane = jax.lax.broadcasted_iota(jnp.int32, (NH * NB, NH), 0)
    whash = jax.lax.broadcasted_iota(jnp.int32, (NH * NB, NH), 1)
    wind = 1 - jnp.minimum(jnp.abs(wlane // NB - whash), 1)
    wsel = ((wlane % NB) * wind).astype(jnp.bfloat16)
    for h in range(H):
        qk_h = qkv[:, DH * h:DH * h + DH]
        bits = jax.lax.bitcast_convert_type(qk_h[:, 0:1], jnp.int32)
        packed = jax.lax.bitcast_convert_type((bits & ~4095) | tcol,
                                              jnp.float32)
        qkvt_ref[h, :, 0:1] = packed
        qkvt_ref[h, :, 1:DH] = qk_h[:, 1:DH]
        qkvt_ref[h, :, DH:2 * DH] = qkv[:, D + DH * h:D + DH * h + DH]
        rot = jax.lax.dot_general(qk_h, rot_ref[...], (((1,), (0,)), ((), ())),
                                  preferred_element_type=jnp.float32)
        parts = [jnp.max(rot[:, NB * hh:NB * hh + NB], axis=1, keepdims=True)
                 for hh in range(NH)]
        m_all = jnp.concatenate(
            [jnp.broadcast_to(p, (TBLK, NB)) for p in parts], axis=1)
        oh = (rot == m_all).astype(jnp.bfloat16)
        idx = jax.lax.dot_general(oh, wsel, (((1,), (0,)), ((), ())),
                                  preferred_element_type=jnp.float32)
        u_ref[h, :, :] = jnp.minimum(idx, float(NB - 1)).astype(jnp.int32)


def _proj_hash(x2, wcat, rot2):
    return pl.pallas_call(
        _proj_hash_body,
        grid=(B * S // TBLK,),
        in_specs=[
            pl.BlockSpec((TBLK, D), lambda i: (i, 0)),
            pl.BlockSpec((2 * D, D), lambda i: (0, 0)),
            pl.BlockSpec((DH, NH * NB), lambda i: (0, 0)),
        ],
        out_specs=[
            pl.BlockSpec((H, TBLK, ROW), lambda i: (i // (S // TBLK),
                                                    i % (S // TBLK), 0)),
            pl.BlockSpec((H, TBLK, NH), lambda i: (i // (S // TBLK),
                                                   i % (S // TBLK), 0)),
        ],
        out_shape=[
            jax.ShapeDtypeStruct((BH, S, ROW), jnp.float32),
            jax.ShapeDtypeStruct((BH, S, NH), jnp.int32),
        ],
    )(x2, wcat, rot2)


def _ranks_body(u_ref, sp_ref):
    bh = pl.program_id(0)
    base = (bh * NI).astype(jnp.float32)
    ub = u_ref[0]
    nbins = NH * NB
    lanes = jax.lax.broadcasted_iota(jnp.int32, (S, nbins), 1)
    ob = (ub[:, 0:1] + 0 * NB) == lanes
    for h in range(1, NH):
        ob = jnp.logical_or(ob, (ub[:, h:h + 1] + h * NB) == lanes)
    obf = ob.astype(jnp.bfloat16)
    of = ob.astype(jnp.float32)
    tri = (jax.lax.broadcasted_iota(jnp.int32, (RANK_C, RANK_C), 0)
           > jax.lax.broadcasted_iota(jnp.int32, (RANK_C, RANK_C), 1))
    trib = tri.astype(jnp.bfloat16)
    ones_row = jnp.ones((1, RANK_C), jnp.bfloat16)
    gsel = (jax.lax.broadcasted_iota(jnp.int32, (nbins, NH), 0) // NB
            == jax.lax.broadcasted_iota(jnp.int32, (nbins, NH), 1)
            ).astype(jnp.bfloat16)
    nchunks = S // RANK_C
    csums = [jax.lax.dot_general(ones_row, obf[RANK_C * k:RANK_C * (k + 1)],
                                 (((1,), (0,)), ((), ())),
                                 preferred_element_type=jnp.float32)
             for k in range(nchunks)]
    hist = csums[0]
    for k in range(1, nchunks):
        hist = hist + csums[k]
    c = hist
    for sft in (1, 2, 4, 8, 16, 32, 64, 128, 256):
        c = c + jnp.concatenate(
            [jnp.zeros((1, sft), jnp.float32), c[:, :nbins - sft]], axis=1)
    goff = c - hist
    run = jnp.zeros((1, nbins), jnp.float32)
    for k in range(nchunks):
        sl = slice(RANK_C * k, RANK_C * (k + 1))
        pk = jax.lax.dot_general(trib, obf[sl], (((1,), (0,)), ((), ())),
                                 preferred_element_type=jnp.float32)
        padj = (pk + (run + goff)) * of[sl]
        hi = jnp.floor(padj * (1.0 / 256.0))
        lo = padj - hi * 256.0
        shi = jax.lax.dot_general(hi.astype(jnp.bfloat16), gsel,
                                  (((1,), (0,)), ((), ())),
                                  preferred_element_type=jnp.float32)
        slo = jax.lax.dot_general(lo.astype(jnp.bfloat16), gsel,
                                  (((1,), (0,)), ((), ())),
                                  preferred_element_type=jnp.float32)
        sp_ref[0, sl, :] = (shi * 256.0 + slo + base).astype(jnp.int32)
        run = run + csums[k]


def _ranks(u):
    return pl.pallas_call(
        _ranks_body,
        grid=(BH,),
        in_specs=[pl.BlockSpec((1, S, NH), lambda i: (i, 0, 0))],
        out_specs=pl.BlockSpec((1, S, NH), lambda i: (i, 0, 0)),
        out_shape=jax.ShapeDtypeStruct((BH, S, NH), jnp.int32),
    )(u)


_SC_W = 128

def _sc_scatter(qkvt_flat, sp_flat):
    mesh = plsc.VectorSubcoreMesh(core_axis_name="core",
                                  subcore_axis_name="subcore")

    @pl.kernel(out_type=jax.ShapeDtypeStruct((TOT, ROW), jnp.float32),
               mesh=mesh, scratch_types=[])
    def k(x_hbm, i_hbm, o_hbm):
        def body(x_vmem, i_vmem):
            pltpu.sync_copy(x_vmem, o_hbm.at[i_vmem.at[0]])

        wins_per_bh = NI // _SC_W
        wins_per_seg = S // _SC_W
        pltpu.emit_pipeline(
            body,
            grid=(TOT // _SC_W,),
            in_specs=[
                pl.BlockSpec((_SC_W, ROW),
                             index_map=lambda w: (
                                 (w // wins_per_bh) * wins_per_seg
                                 + w % wins_per_seg, 0)),
                pl.BlockSpec((1, _SC_W), index_map=lambda w: (0, w)),
            ],
            out_specs=[],
            core_axis_name=("core", "subcore"),
            dimension_semantics=(pltpu.PARALLEL,),
        )(x_hbm, i_hbm)

    return k(qkvt_flat, sp_flat)


def _sc_gather(so_flat, sp_flat):
    mesh = plsc.VectorSubcoreMesh(core_axis_name="core",
                                  subcore_axis_name="subcore")

    @pl.kernel(out_type=jax.ShapeDtypeStruct((TOT, OROW), jnp.float32),
               mesh=mesh, scratch_types=[])
    def k(x_hbm, i_hbm, o_hbm):
        def body(i_vmem, o_vmem):
            pltpu.sync_copy(x_hbm.at[i_vmem.at[0]], o_vmem)

        pltpu.emit_pipeline(
            body,
            grid=(TOT // _SC_W,),
            in_specs=[pl.BlockSpec((1, _SC_W), index_map=lambda w: (0, w))],
            out_specs=[pl.BlockSpec((_SC_W, OROW),
                                    index_map=lambda w: (w, 0))],
            core_axis_name=("core", "subcore"),
            dimension_semantics=(pltpu.PARALLEL,),
        )(i_hbm, o_hbm)

    return k(so_flat, sp_flat)


def _attn_body(cur_ref, prev_ref, tbnd_ref, so_ref):
    cb = pl.program_id(1)
    is_bnd = (cb % (NCHUNK // NH // CHUNK_BLK)) == 0
    nq = CHUNK_BLK * NB
    cur = cur_ref[0]
    prev = prev_ref[0]
    kc = cur[:, 0:DH]
    kp = prev[:, 0:DH]
    bk_c = (kc / jnp.maximum(
        jnp.sqrt(jnp.sum(kc * kc, axis=1, keepdims=True)), 1e-12)
            ).astype(jnp.bfloat16)
    bk_p = (kp / jnp.maximum(
        jnp.sqrt(jnp.sum(kp * kp, axis=1, keepdims=True)), 1e-12)
            ).astype(jnp.bfloat16)
    bk_all = jnp.concatenate([bk_p, bk_c], axis=0)
    v_all = jnp.concatenate([prev[:, DH:2 * DH],
                             cur[:, DH:2 * DH]], axis=0).astype(jnp.bfloat16)
    q = kc.astype(jnp.bfloat16)
    dots_b = jax.lax.dot_general(q, bk_all, (((1,), (1,)), ((), ())),
                                 preferred_element_type=jnp.float32)
    dots = jnp.concatenate(
        [dots_b[NB * k:NB * (k + 1), NB * k:NB * k + 2 * NB]
         for k in range(CHUNK_BLK)], axis=0)
    dots = dots * (DH ** -0.5)
    ri = jax.lax.broadcasted_iota(jnp.int32, (nq, 1), 0)
    ci = jax.lax.broadcasted_iota(jnp.int32, (1, 2 * NB), 1)
    mask = ci == (ri % NB + NB)
    tk_prev = tbnd_ref[0, 0]
    tk128 = jnp.concatenate(
        [tk_prev, jnp.full((1, NB), jnp.inf, jnp.float32)], axis=1)
    same_tok = jnp.logical_and(cur[:, 0:1] == tk128,
                               jnp.logical_and(ri < NB, is_bnd))
    dots = jnp.where(jnp.logical_or(mask, same_tok), SELF_VAL, dots)
    m = jnp.max(dots, axis=1, keepdims=True)
    e = jnp.exp(dots - m)
    ssum = jnp.sum(e, axis=1, keepdims=True)
    lse = m + jnp.log(ssum)
    probs = (e / ssum).astype(jnp.bfloat16)
    for k in range(CHUNK_BLK):
        bo = jax.lax.dot_general(probs[NB * k:NB * (k + 1)],
                                 v_all[NB * k:NB * k + 2 * NB],
                                 (((1,), (0,)), ((), ())),
                                 preferred_element_type=jnp.float32)
        so_ref[0, NB * k:NB * (k + 1), 0:DH] = bo
    so_ref[0, :, DH:DH + 1] = lse


def _attention(sorted3, t_bnd):
    rows_blk = CHUNK_BLK * NB
    return pl.pallas_call(
        _attn_body,
        grid=(BH, NCHUNK // CHUNK_BLK),
        in_specs=[
            pl.BlockSpec((1, rows_blk, ROW), lambda bh, cb: (bh, cb, 0)),
            pl.BlockSpec((1, NB, ROW),
                         lambda bh, cb: (bh, (cb * CHUNK_BLK + NCHUNK - 1)
                                         % NCHUNK, 0)),
            pl.BlockSpec((1, 1, 1, NB),
                         lambda bh, cb: (bh, (cb // (NCHUNK // NH // CHUNK_BLK)
                                              + NH - 1) % NH, 0, 0)),
        ],
        out_specs=pl.BlockSpec((1, rows_blk, OROW),
                               lambda bh, cb: (bh, cb, 0)),
        out_shape=jax.ShapeDtypeStruct((BH, NI, OROW), jnp.float32),
    )(sorted3, sorted3, t_bnd)


def _combine_body(g_ref, wout_ref, bout_ref, out_ref):
    combs = []
    for h in range(H):
        gh = g_ref[h]
        logits = gh[:, :, DH:DH + 1]
        m = jnp.max(logits, axis=0)
        e = jnp.exp(logits - m[None])
        ssum = jnp.sum(e, axis=0)
        probs = e / ssum[None]
        combs.append(jnp.sum(gh[:, :, 0:DH] * probs, axis=0))
    out_tok = jnp.concatenate(combs, axis=1)
    res = jax.lax.dot_general(out_tok, wout_ref[...], (((1,), (1,)), ((), ())),
                              preferred_element_type=jnp.float32)
    out_ref[0] = res + bout_ref[...]


def _combine(gathered, wout, bout2):
    return pl.pallas_call(
        _combine_body,
        grid=(B, S // TBLK),
        in_specs=[
            pl.BlockSpec((H, NH, TBLK, OROW), lambda b, tb: (b, 0, tb, 0)),
            pl.BlockSpec((D, D), lambda b, tb: (0, 0)),
            pl.BlockSpec((1, D), lambda b, tb: (0, 0)),
        ],
        out_specs=pl.BlockSpec((1, TBLK, D), lambda b, tb: (b, tb, 0)),
        out_shape=jax.ShapeDtypeStruct((B, S, D), jnp.float32),
    )(gathered, wout, bout2)


def kernel(x, Wqk, Wv, Wout, bout, rotations):
    x2 = x.reshape(B * S, D)
    wcat = jnp.concatenate([Wqk, Wv], axis=0)
    rot2 = jnp.concatenate([rotations, -rotations], axis=2).reshape(DH, NH * NB)

    qkvt, u = _proj_hash(x2, wcat, rot2)
    sp = _ranks(u)
    sp_flat = sp.transpose(0, 2, 1).reshape(1, TOT)

    sorted_flat = _sc_scatter(qkvt.reshape(BH * S, ROW), sp_flat)
    sorted3 = sorted_flat.reshape(BH, NI, ROW)
    t_bnd = (sorted3.reshape(BH, NH, NCHUNK // NH, NB, ROW)
             [:, :, NCHUNK // NH - 1, :, 0].reshape(BH, NH, 1, NB))

    so = _attention(sorted3, t_bnd)
    gathered = _sc_gather(so.reshape(TOT, OROW), sp_flat)
    g4 = gathered.reshape(BH, NH, S, OROW)

    return _combine(g4, Wout, bout.reshape(1, D))
